# Initial kernel scaffold; baseline (speedup 1.0000x reference)
import functools

import jax
import jax.numpy as jnp
from jax import lax
from jax.experimental import pallas as pl
from jax.experimental.pallas import tpu as pltpu
from jax.experimental.pallas import tpu_sc as plsc

N_NODES = 10000
N_EDGES = 320000
HID = 128
PADW = 144

NC = 2
NS = 16
NW = NC * NS
CHUNK = 80
EDGES_PER_W = N_EDGES // NW
CHUNKS_PER_W = EDGES_PER_W // CHUNK
ROWS_PER_SUB = N_NODES // NS

BR = 1250
GRID = N_NODES // BR

_HIGHEST = lax.Precision.HIGHEST


def _prep_body(x_ref, h_ref, wxt_ref, wht_ref, bp_ref, whht_ref, bhh_ref,
               mpad_ref, gh_ref):
    xb = x_ref[...]
    hb = h_ref[...]
    mpad_ref[...] = (
        jnp.dot(xb, wxt_ref[...], precision=_HIGHEST)
        + jnp.dot(hb, wht_ref[...], precision=_HIGHEST)
        + bp_ref[...]
    )
    gh_ref[...] = jnp.dot(hb, whht_ref[...], precision=_HIGHEST) + bhh_ref[...]


def _prep(x, h, wxt_pad, wht_pad, b_pad, whht, bhh):
    return pl.pallas_call(
        _prep_body,
        grid=(GRID,),
        in_specs=[
            pl.BlockSpec((BR, HID), lambda i: (i, 0)),
            pl.BlockSpec((BR, HID), lambda i: (i, 0)),
            pl.BlockSpec((HID, PADW), lambda i: (0, 0)),
            pl.BlockSpec((HID, PADW), lambda i: (0, 0)),
            pl.BlockSpec((1, PADW), lambda i: (0, 0)),
            pl.BlockSpec((HID, 3 * HID), lambda i: (0, 0)),
            pl.BlockSpec((1, 3 * HID), lambda i: (0, 0)),
        ],
        out_specs=[
            pl.BlockSpec((BR, PADW), lambda i: (i, 0)),
            pl.BlockSpec((BR, 3 * HID), lambda i: (i, 0)),
        ],
        out_shape=[
            jax.ShapeDtypeStruct((N_NODES, PADW), jnp.float32),
            jax.ShapeDtypeStruct((N_NODES, 3 * HID), jnp.float32),
        ],
    )(x, h, wxt_pad, wht_pad, b_pad, whht, bhh)


def _seg_body(mpad_hbm, src_hbm, dst_hbm, zeros_hbm, out_hbm,
              idx_s, idx_d, rows, acc, sem):
    cid = lax.axis_index("c")
    sid = lax.axis_index("s")
    wid = cid * NS + sid

    pltpu.sync_copy(zeros_hbm, acc.at[pl.ds(sid * ROWS_PER_SUB, ROWS_PER_SUB)])
    plsc.subcore_barrier()

    edge_base = pl.multiple_of(wid * EDGES_PER_W, 8)

    def body(j):
        base = pl.multiple_of(edge_base + j * CHUNK, 8)
        pltpu.sync_copy(src_hbm.at[pl.ds(base, CHUNK)], idx_s)
        pltpu.sync_copy(dst_hbm.at[pl.ds(base, CHUNK)], idx_d)
        pltpu.async_copy(mpad_hbm.at[idx_s], rows, sem).wait()
        pltpu.sync_copy(rows, acc.at[idx_d], add=True)

    pl.loop(0, CHUNKS_PER_W)(body)
    plsc.subcore_barrier()

    pltpu.sync_copy(
        acc.at[pl.ds(sid * ROWS_PER_SUB, ROWS_PER_SUB)],
        out_hbm.at[cid, pl.ds(sid * ROWS_PER_SUB, ROWS_PER_SUB)],
    )


def _segment_accumulate(mpad, src, dst, zeros):
    mesh = plsc.VectorSubcoreMesh(core_axis_name="c", subcore_axis_name="s")
    return pl.kernel(
        _seg_body,
        out_type=jax.ShapeDtypeStruct((NC, N_NODES, PADW), jnp.float32),
        mesh=mesh,
        scratch_types=[
            pltpu.VMEM((CHUNK,), jnp.int32),
            pltpu.VMEM((CHUNK,), jnp.int32),
            pltpu.VMEM((CHUNK, PADW), jnp.float32),
            pltpu.VMEM_SHARED((N_NODES, PADW), jnp.float32),
            pltpu.SemaphoreType.DMA,
        ],
    )(mpad, src, dst, zeros)


def _gru_body(x_ref, h_ref, part_ref, gh_ref, wixt_ref, wict_ref, bih_ref,
              out_ref):
    s = part_ref[0] + part_ref[1]
    cnt = jnp.maximum(s[:, HID:HID + 1], 1.0)
    c = s[:, :HID] / cnt
    xb = x_ref[...]
    hb = h_ref[...]
    gi = (
        jnp.dot(xb, wixt_ref[...], precision=_HIGHEST)
        + jnp.dot(c, wict_ref[...], precision=_HIGHEST)
        + bih_ref[...]
    )
    gh = gh_ref[...]
    r = jax.nn.sigmoid(gi[:, :HID] + gh[:, :HID])
    z = jax.nn.sigmoid(gi[:, HID:2 * HID] + gh[:, HID:2 * HID])
    nn_ = jnp.tanh(gi[:, 2 * HID:] + r * gh[:, 2 * HID:])
    out_ref[...] = (1.0 - z) * nn_ + z * hb


def _gru(x, h, partials, gh, wixt, wict, bih):
    return pl.pallas_call(
        _gru_body,
        grid=(GRID,),
        in_specs=[
            pl.BlockSpec((BR, HID), lambda i: (i, 0)),
            pl.BlockSpec((BR, HID), lambda i: (i, 0)),
            pl.BlockSpec((NC, BR, PADW), lambda i: (0, i, 0)),
            pl.BlockSpec((BR, 3 * HID), lambda i: (i, 0)),
            pl.BlockSpec((HID, 3 * HID), lambda i: (0, 0)),
            pl.BlockSpec((HID, 3 * HID), lambda i: (0, 0)),
            pl.BlockSpec((1, 3 * HID), lambda i: (0, 0)),
        ],
        out_specs=pl.BlockSpec((BR, HID), lambda i: (i, 0)),
        out_shape=jax.ShapeDtypeStruct((N_NODES, HID), jnp.float32),
    )(x, h, partials, gh, wixt, wict, bih)


def kernel(x, h, edge_index, W_msg, b_msg, W_ih, W_hh, b_ih, b_hh):
    src = edge_index[0].astype(jnp.int32)
    dst = edge_index[1].astype(jnp.int32)

    wxt_pad = jnp.pad(W_msg[:, :HID].T, ((0, 0), (0, PADW - HID)))
    wht_pad = jnp.pad(W_msg[:, HID:].T, ((0, 0), (0, PADW - HID)))
    b_pad = jnp.pad(b_msg, (0, PADW - HID)).at[HID].set(1.0)[None, :]

    whht = W_hh.T
    bhh = b_hh[None, :]
    wixt = W_ih[:, :HID].T
    wict = W_ih[:, HID:].T
    bih = b_ih[None, :]

    zeros = jnp.zeros((ROWS_PER_SUB, PADW), jnp.float32)

    mpad, gh = _prep(x, h, wxt_pad, wht_pad, b_pad, whht, bhh)
    partials = _segment_accumulate(mpad, src, dst, zeros)
    return _gru(x, h, partials, gh, wixt, wict, bih)


# speedup vs baseline: 10.1354x; 10.1354x over previous
import functools

import jax
import jax.numpy as jnp
from jax import lax
from jax.experimental import pallas as pl
from jax.experimental.pallas import tpu as pltpu
from jax.experimental.pallas import tpu_sc as plsc

N_NODES = 10000
N_EDGES = 320000
HID = 128
PADW = 144

NC = 2
NS = 16
NW = NC * NS
CHUNK = 40
EDGES_PER_W = N_EDGES // NW
CHUNKS_PER_W = EDGES_PER_W // CHUNK
N_PAD = 10240
ROWS_PER_SUB = N_PAD // NS

BR = 2000
GRID = N_NODES // BR

_HIGHEST = lax.Precision.HIGHEST


def _prep_body(x_ref, h_ref, wxt_ref, wht_ref, bp_ref, whht_ref, bhh_ref,
               mpad_ref, gh_ref):
    xb = x_ref[...]
    hb = h_ref[...]
    mpad_ref[...] = (
        jnp.dot(xb, wxt_ref[...], precision=_HIGHEST)
        + jnp.dot(hb, wht_ref[...], precision=_HIGHEST)
        + bp_ref[...]
    )
    gh_ref[...] = jnp.dot(hb, whht_ref[...], precision=_HIGHEST) + bhh_ref[...]


def _prep(x, h, wxt_pad, wht_pad, b_pad, whht, bhh):
    return pl.pallas_call(
        _prep_body,
        grid=(GRID,),
        in_specs=[
            pl.BlockSpec((BR, HID), lambda i: (i, 0)),
            pl.BlockSpec((BR, HID), lambda i: (i, 0)),
            pl.BlockSpec((HID, PADW), lambda i: (0, 0)),
            pl.BlockSpec((HID, PADW), lambda i: (0, 0)),
            pl.BlockSpec((1, PADW), lambda i: (0, 0)),
            pl.BlockSpec((HID, 3 * HID), lambda i: (0, 0)),
            pl.BlockSpec((1, 3 * HID), lambda i: (0, 0)),
        ],
        out_specs=[
            pl.BlockSpec((BR, PADW), lambda i: (i, 0)),
            pl.BlockSpec((BR, 3 * HID), lambda i: (i, 0)),
        ],
        out_shape=[
            jax.ShapeDtypeStruct((N_NODES, PADW), jnp.float32),
            jax.ShapeDtypeStruct((N_NODES, 3 * HID), jnp.float32),
        ],
    )(x, h, wxt_pad, wht_pad, b_pad, whht, bhh)


def _seg_body(mpad_hbm, src2_hbm, dst2_hbm, zeros_hbm, out_hbm,
              src_t, dst_t, rows0, rows1, acc, sem0, sem1):
    cid = lax.axis_index("c")
    sid = lax.axis_index("s")
    wid = cid * NS + sid

    pltpu.sync_copy(zeros_hbm, acc.at[pl.ds(sid * ROWS_PER_SUB, ROWS_PER_SUB)])
    row0 = wid * CHUNKS_PER_W
    pltpu.sync_copy(src2_hbm.at[pl.ds(row0, CHUNKS_PER_W)], src_t)
    pltpu.sync_copy(dst2_hbm.at[pl.ds(row0, CHUNKS_PER_W)], dst_t)
    plsc.subcore_barrier()

    def start(j, buf, sem):
        pltpu.async_copy(mpad_hbm.at[src_t.at[j]], buf, sem)

    def wait(buf, sem):
        pltpu.make_async_copy(mpad_hbm.at[src_t.at[0]], buf, sem).wait()

    def scat(j, buf):
        pltpu.sync_copy(buf, acc.at[dst_t.at[j]], add=True)

    start(0, rows0, sem0)
    start(1, rows1, sem1)

    def body(j):
        wait(rows0, sem0)
        scat(j, rows0)
        start(j + 2, rows0, sem0)
        wait(rows1, sem1)
        scat(j + 1, rows1)
        start(j + 3, rows1, sem1)

    pl.loop(0, CHUNKS_PER_W - 3, step=2)(body)
    wait(rows0, sem0)
    scat(CHUNKS_PER_W - 3, rows0)
    start(CHUNKS_PER_W - 1, rows0, sem0)
    wait(rows1, sem1)
    scat(CHUNKS_PER_W - 2, rows1)
    wait(rows0, sem0)
    scat(CHUNKS_PER_W - 1, rows0)

    plsc.subcore_barrier()

    pltpu.sync_copy(
        acc.at[pl.ds(sid * ROWS_PER_SUB, ROWS_PER_SUB)],
        out_hbm.at[cid, pl.ds(sid * ROWS_PER_SUB, ROWS_PER_SUB)],
    )


def _segment_accumulate(mpad, src2, dst2, zeros):
    mesh = plsc.VectorSubcoreMesh(core_axis_name="c", subcore_axis_name="s")
    return pl.kernel(
        _seg_body,
        out_type=jax.ShapeDtypeStruct((NC, N_PAD, PADW), jnp.float32),
        mesh=mesh,
        compiler_params=pltpu.CompilerParams(use_tc_tiling_on_sc=False),
        scratch_types=[
            pltpu.VMEM((CHUNKS_PER_W, CHUNK), jnp.int32),
            pltpu.VMEM((CHUNKS_PER_W, CHUNK), jnp.int32),
            pltpu.VMEM((CHUNK, PADW), jnp.float32),
            pltpu.VMEM((CHUNK, PADW), jnp.float32),
            pltpu.VMEM_SHARED((N_PAD, PADW), jnp.float32),
            pltpu.SemaphoreType.DMA,
            pltpu.SemaphoreType.DMA,
        ],
    )(mpad, src2, dst2, zeros)


def _gru_body(x_ref, h_ref, part_ref, gh_ref, wixt_ref, wict_ref, bih_ref,
              out_ref):
    s = part_ref[0] + part_ref[1]
    cnt = jnp.maximum(s[:, HID:HID + 1], 1.0)
    c = s[:, :HID] / cnt
    xb = x_ref[...]
    hb = h_ref[...]
    gi = (
        jnp.dot(xb, wixt_ref[...], precision=_HIGHEST)
        + jnp.dot(c, wict_ref[...], precision=_HIGHEST)
        + bih_ref[...]
    )
    gh = gh_ref[...]
    r = jax.nn.sigmoid(gi[:, :HID] + gh[:, :HID])
    z = jax.nn.sigmoid(gi[:, HID:2 * HID] + gh[:, HID:2 * HID])
    nn_ = jnp.tanh(gi[:, 2 * HID:] + r * gh[:, 2 * HID:])
    out_ref[...] = (1.0 - z) * nn_ + z * hb


def _gru(x, h, partials, gh, wixt, wict, bih):
    return pl.pallas_call(
        _gru_body,
        grid=(GRID,),
        in_specs=[
            pl.BlockSpec((BR, HID), lambda i: (i, 0)),
            pl.BlockSpec((BR, HID), lambda i: (i, 0)),
            pl.BlockSpec((NC, BR, PADW), lambda i: (0, i, 0)),
            pl.BlockSpec((BR, 3 * HID), lambda i: (i, 0)),
            pl.BlockSpec((HID, 3 * HID), lambda i: (0, 0)),
            pl.BlockSpec((HID, 3 * HID), lambda i: (0, 0)),
            pl.BlockSpec((1, 3 * HID), lambda i: (0, 0)),
        ],
        out_specs=pl.BlockSpec((BR, HID), lambda i: (i, 0)),
        out_shape=jax.ShapeDtypeStruct((N_NODES, HID), jnp.float32),
    )(x, h, partials, gh, wixt, wict, bih)


def kernel(x, h, edge_index, W_msg, b_msg, W_ih, W_hh, b_ih, b_hh):
    src2 = edge_index[0].astype(jnp.int32).reshape(N_EDGES // CHUNK, CHUNK)
    dst2 = edge_index[1].astype(jnp.int32).reshape(N_EDGES // CHUNK, CHUNK)

    wxt_pad = jnp.pad(W_msg[:, :HID].T, ((0, 0), (0, PADW - HID)))
    wht_pad = jnp.pad(W_msg[:, HID:].T, ((0, 0), (0, PADW - HID)))
    b_pad = jnp.pad(b_msg, (0, PADW - HID)).at[HID].set(1.0)[None, :]

    whht = W_hh.T
    bhh = b_hh[None, :]
    wixt = W_ih[:, :HID].T
    wict = W_ih[:, HID:].T
    bih = b_ih[None, :]

    zeros = jnp.zeros((ROWS_PER_SUB, PADW), jnp.float32)

    mpad, gh = _prep(x, h, wxt_pad, wht_pad, b_pad, whht, bhh)
    partials = _segment_accumulate(mpad, src2, dst2, zeros)
    return _gru(x, h, partials, gh, wixt, wict, bih)


# speedup vs baseline: 10.1380x; 1.0003x over previous
import functools

import jax
import jax.numpy as jnp
from jax import lax
from jax.experimental import pallas as pl
from jax.experimental.pallas import tpu as pltpu
from jax.experimental.pallas import tpu_sc as plsc

N_NODES = 10000
N_EDGES = 320000
HID = 128
PADW = 144

NC = 2
NS = 16
NW = NC * NS
CHUNK = 40
EDGES_PER_W = N_EDGES // NW
CHUNKS_PER_W = EDGES_PER_W // CHUNK
N_PAD = 10240
ROWS_PER_SUB = N_PAD // NS

BR = 2000
GRID = N_NODES // BR

_HIGHEST = lax.Precision.HIGHEST


def _prep_body(x_ref, h_ref, wxt_ref, wht_ref, bp_ref, whht_ref, bhh_ref,
               mpad_ref, gh_ref):
    xb = x_ref[...]
    hb = h_ref[...]
    mpad_ref[...] = (
        jnp.dot(xb, wxt_ref[...], precision=_HIGHEST)
        + jnp.dot(hb, wht_ref[...], precision=_HIGHEST)
        + bp_ref[...]
    )
    gh_ref[...] = jnp.dot(hb, whht_ref[...], precision=_HIGHEST) + bhh_ref[...]


def _prep(x, h, wxt_pad, wht_pad, b_pad, whht, bhh):
    return pl.pallas_call(
        _prep_body,
        grid=(GRID,),
        in_specs=[
            pl.BlockSpec((BR, HID), lambda i: (i, 0)),
            pl.BlockSpec((BR, HID), lambda i: (i, 0)),
            pl.BlockSpec((HID, PADW), lambda i: (0, 0)),
            pl.BlockSpec((HID, PADW), lambda i: (0, 0)),
            pl.BlockSpec((1, PADW), lambda i: (0, 0)),
            pl.BlockSpec((HID, 3 * HID), lambda i: (0, 0)),
            pl.BlockSpec((1, 3 * HID), lambda i: (0, 0)),
        ],
        out_specs=[
            pl.BlockSpec((BR, PADW), lambda i: (i, 0)),
            pl.BlockSpec((BR, 3 * HID), lambda i: (i, 0)),
        ],
        out_shape=[
            jax.ShapeDtypeStruct((N_NODES, PADW), jnp.float32),
            jax.ShapeDtypeStruct((N_NODES, 3 * HID), jnp.float32),
        ],
    )(x, h, wxt_pad, wht_pad, b_pad, whht, bhh)


def _seg_body(mpad_hbm, src2_hbm, dst2_hbm, zeros_hbm, out_hbm,
              src_t, dst_t, rows0, rows1, acc, sem0, sem1):
    cid = lax.axis_index("c")
    sid = lax.axis_index("s")
    wid = cid * NS + sid

    pltpu.sync_copy(zeros_hbm, acc.at[pl.ds(sid * ROWS_PER_SUB, ROWS_PER_SUB)])
    row0 = wid * CHUNKS_PER_W
    pltpu.sync_copy(src2_hbm.at[pl.ds(row0, CHUNKS_PER_W)], src_t)
    pltpu.sync_copy(dst2_hbm.at[pl.ds(row0, CHUNKS_PER_W)], dst_t)
    plsc.subcore_barrier()

    def start(j, buf, sem):
        pltpu.async_copy(mpad_hbm.at[src_t.at[j]], buf, sem)

    def wait(buf, sem):
        pltpu.make_async_copy(mpad_hbm.at[src_t.at[0]], buf, sem).wait()

    def scat(j, buf):
        pltpu.sync_copy(buf, acc.at[dst_t.at[j]], add=True)

    start(0, rows0, sem0)
    start(1, rows1, sem1)

    def body(j):
        wait(rows0, sem0)
        scat(j, rows0)
        start(j + 2, rows0, sem0)
        wait(rows1, sem1)
        scat(j + 1, rows1)
        start(j + 3, rows1, sem1)

    pl.loop(0, CHUNKS_PER_W - 2, step=2)(body)
    wait(rows0, sem0)
    scat(CHUNKS_PER_W - 2, rows0)
    wait(rows1, sem1)
    scat(CHUNKS_PER_W - 1, rows1)

    plsc.subcore_barrier()

    pltpu.sync_copy(
        acc.at[pl.ds(sid * ROWS_PER_SUB, ROWS_PER_SUB)],
        out_hbm.at[cid, pl.ds(sid * ROWS_PER_SUB, ROWS_PER_SUB)],
    )


def _segment_accumulate(mpad, src2, dst2, zeros):
    mesh = plsc.VectorSubcoreMesh(core_axis_name="c", subcore_axis_name="s")
    return pl.kernel(
        _seg_body,
        out_type=jax.ShapeDtypeStruct((NC, N_PAD, PADW), jnp.float32),
        mesh=mesh,
        compiler_params=pltpu.CompilerParams(use_tc_tiling_on_sc=False),
        scratch_types=[
            pltpu.VMEM((CHUNKS_PER_W, CHUNK), jnp.int32),
            pltpu.VMEM((CHUNKS_PER_W, CHUNK), jnp.int32),
            pltpu.VMEM((CHUNK, PADW), jnp.float32),
            pltpu.VMEM((CHUNK, PADW), jnp.float32),
            pltpu.VMEM_SHARED((N_PAD, PADW), jnp.float32),
            pltpu.SemaphoreType.DMA,
            pltpu.SemaphoreType.DMA,
        ],
    )(mpad, src2, dst2, zeros)


def _gru_body(x_ref, h_ref, part_ref, gh_ref, wixt_ref, wict_ref, bih_ref,
              out_ref):
    s = part_ref[0] + part_ref[1]
    cnt = jnp.maximum(s[:, HID:HID + 1], 1.0)
    c = s[:, :HID] / cnt
    xb = x_ref[...]
    hb = h_ref[...]
    gi = (
        jnp.dot(xb, wixt_ref[...], precision=_HIGHEST)
        + jnp.dot(c, wict_ref[...], precision=_HIGHEST)
        + bih_ref[...]
    )
    gh = gh_ref[...]
    r = jax.nn.sigmoid(gi[:, :HID] + gh[:, :HID])
    z = jax.nn.sigmoid(gi[:, HID:2 * HID] + gh[:, HID:2 * HID])
    nn_ = jnp.tanh(gi[:, 2 * HID:] + r * gh[:, 2 * HID:])
    out_ref[...] = (1.0 - z) * nn_ + z * hb


def _gru(x, h, partials, gh, wixt, wict, bih):
    return pl.pallas_call(
        _gru_body,
        grid=(GRID,),
        in_specs=[
            pl.BlockSpec((BR, HID), lambda i: (i, 0)),
            pl.BlockSpec((BR, HID), lambda i: (i, 0)),
            pl.BlockSpec((NC, BR, PADW), lambda i: (0, i, 0)),
            pl.BlockSpec((BR, 3 * HID), lambda i: (i, 0)),
            pl.BlockSpec((HID, 3 * HID), lambda i: (0, 0)),
            pl.BlockSpec((HID, 3 * HID), lambda i: (0, 0)),
            pl.BlockSpec((1, 3 * HID), lambda i: (0, 0)),
        ],
        out_specs=pl.BlockSpec((BR, HID), lambda i: (i, 0)),
        out_shape=jax.ShapeDtypeStruct((N_NODES, HID), jnp.float32),
    )(x, h, partials, gh, wixt, wict, bih)


def kernel(x, h, edge_index, W_msg, b_msg, W_ih, W_hh, b_ih, b_hh):
    src2 = edge_index[0].astype(jnp.int32).reshape(N_EDGES // CHUNK, CHUNK)
    dst2 = edge_index[1].astype(jnp.int32).reshape(N_EDGES // CHUNK, CHUNK)

    wxt_pad = jnp.pad(W_msg[:, :HID].T, ((0, 0), (0, PADW - HID)))
    wht_pad = jnp.pad(W_msg[:, HID:].T, ((0, 0), (0, PADW - HID)))
    b_pad = jnp.pad(b_msg, (0, PADW - HID)).at[HID].set(1.0)[None, :]

    whht = W_hh.T
    bhh = b_hh[None, :]
    wixt = W_ih[:, :HID].T
    wict = W_ih[:, HID:].T
    bih = b_ih[None, :]

    zeros = jnp.zeros((ROWS_PER_SUB, PADW), jnp.float32)

    mpad, gh = _prep(x, h, wxt_pad, wht_pad, b_pad, whht, bhh)
    partials = _segment_accumulate(mpad, src2, dst2, zeros)
    return _gru(x, h, partials, gh, wixt, wict, bih)


# speedup vs baseline: 16.9613x; 1.6730x over previous
import functools

import jax
import jax.numpy as jnp
from jax import lax
from jax.experimental import pallas as pl
from jax.experimental.pallas import tpu as pltpu
from jax.experimental.pallas import tpu_sc as plsc

N_NODES = 10000
N_EDGES = 320000
HID = 128

NC = 2
NS = 16
NW = NC * NS
CHUNK = 80
LANES = 16
EDGES_PER_W = N_EDGES // NW
CHUNKS_PER_W = EDGES_PER_W // CHUNK
N_PAD = 10240
ROWS_PER_SUB = N_PAD // NS

BR = 2000
GRID = N_NODES // BR
GBR = 2048
GGRID = N_PAD // GBR


def _prep_body(x_ref, h_ref, wxt_ref, wht_ref, bm_ref, m_ref):
    m_ref[...] = (
        jnp.dot(x_ref[...], wxt_ref[...])
        + jnp.dot(h_ref[...], wht_ref[...])
        + bm_ref[...]
    )


def _prep(x, h, wxt, wht, bm):
    return pl.pallas_call(
        _prep_body,
        grid=(GRID,),
        in_specs=[
            pl.BlockSpec((BR, HID), lambda i: (i, 0)),
            pl.BlockSpec((BR, HID), lambda i: (i, 0)),
            pl.BlockSpec((HID, HID), lambda i: (0, 0)),
            pl.BlockSpec((HID, HID), lambda i: (0, 0)),
            pl.BlockSpec((1, HID), lambda i: (0, 0)),
        ],
        out_specs=pl.BlockSpec((BR, HID), lambda i: (i, 0)),
        out_shape=jax.ShapeDtypeStruct((N_NODES, HID), jnp.float32),
    )(x, h, wxt, wht, bm)


def _seg_body(m_hbm, pk_hbm, za_hbm, zc_hbm, sums_hbm, cnts_hbm,
              pk_t, s0, d0, s1, d1, rows0, rows1, cnt, acc, sem0, sem1):
    cid = lax.axis_index("c")
    sid = lax.axis_index("s")
    wid = cid * NS + sid

    pltpu.sync_copy(za_hbm, acc.at[pl.ds(sid * ROWS_PER_SUB, ROWS_PER_SUB)])
    pltpu.sync_copy(zc_hbm, cnt)
    base = pl.multiple_of(wid * EDGES_PER_W, 8)
    pltpu.sync_copy(pk_hbm.at[pl.ds(base, EDGES_PER_W)], pk_t)
    plsc.subcore_barrier()

    ones = jnp.full((LANES,), 1.0, jnp.float32)

    def unpack(j, s_ref, d_ref):
        for v in range(CHUNK // LANES):
            pk = pk_t[pl.ds(j * CHUNK + v * LANES, LANES)]
            dvec = lax.bitwise_and(pk, 16383)
            s_ref[pl.ds(v * LANES, LANES)] = lax.shift_right_logical(pk, 14)
            d_ref[pl.ds(v * LANES, LANES)] = dvec
            plsc.addupdate_scatter(cnt, [dvec], ones)

    def start(s_ref, buf, sem):
        pltpu.async_copy(m_hbm.at[s_ref], buf, sem)

    def wait(buf, sem):
        pltpu.make_async_copy(m_hbm.at[s0], buf, sem).wait()

    def scat(d_ref, buf):
        pltpu.sync_copy(buf, acc.at[d_ref], add=True)

    unpack(0, s0, d0)
    start(s0, rows0, sem0)
    unpack(1, s1, d1)
    start(s1, rows1, sem1)

    def body(j):
        wait(rows0, sem0)
        scat(d0, rows0)
        unpack(j + 2, s0, d0)
        start(s0, rows0, sem0)
        wait(rows1, sem1)
        scat(d1, rows1)
        unpack(j + 3, s1, d1)
        start(s1, rows1, sem1)

    pl.loop(0, CHUNKS_PER_W - 3, step=2)(body)
    wait(rows0, sem0)
    scat(d0, rows0)
    unpack(CHUNKS_PER_W - 1, s0, d0)
    start(s0, rows0, sem0)
    wait(rows1, sem1)
    scat(d1, rows1)
    wait(rows0, sem0)
    scat(d0, rows0)

    plsc.subcore_barrier()

    pltpu.sync_copy(
        acc.at[pl.ds(sid * ROWS_PER_SUB, ROWS_PER_SUB)],
        sums_hbm.at[cid, pl.ds(sid * ROWS_PER_SUB, ROWS_PER_SUB)],
    )
    pltpu.sync_copy(cnt, cnts_hbm.at[cid, sid])


def _segment_accumulate(m, pk, za, zc):
    mesh = plsc.VectorSubcoreMesh(core_axis_name="c", subcore_axis_name="s")
    return pl.kernel(
        _seg_body,
        out_type=[
            jax.ShapeDtypeStruct((NC, N_PAD, HID), jnp.float32),
            jax.ShapeDtypeStruct((NC, NS, N_PAD), jnp.float32),
        ],
        mesh=mesh,
        compiler_params=pltpu.CompilerParams(needs_layout_passes=False),
        scratch_types=[
            pltpu.VMEM((EDGES_PER_W,), jnp.int32),
            pltpu.VMEM((CHUNK,), jnp.int32),
            pltpu.VMEM((CHUNK,), jnp.int32),
            pltpu.VMEM((CHUNK,), jnp.int32),
            pltpu.VMEM((CHUNK,), jnp.int32),
            pltpu.VMEM((CHUNK, HID), jnp.float32),
            pltpu.VMEM((CHUNK, HID), jnp.float32),
            pltpu.VMEM((N_PAD,), jnp.float32),
            pltpu.VMEM_SHARED((N_PAD, HID), jnp.float32),
            pltpu.SemaphoreType.DMA,
            pltpu.SemaphoreType.DMA,
        ],
    )(m, pk, za, zc)


def _gru_body(x_ref, h_ref, part_ref, cnt_ref, wixt_ref, wict_ref, whht_ref,
              bih_ref, bhh_ref, out_ref):
    s = part_ref[0] + part_ref[1]
    n_in = jnp.sum(cnt_ref[...], axis=(0, 1))
    c = s / jnp.maximum(n_in, 1.0)[:, None]
    xb = x_ref[...]
    hb = h_ref[...]
    gi = (
        jnp.dot(xb, wixt_ref[...])
        + jnp.dot(c, wict_ref[...])
        + bih_ref[...]
    )
    gh = jnp.dot(hb, whht_ref[...]) + bhh_ref[...]
    r = jax.nn.sigmoid(gi[:, :HID] + gh[:, :HID])
    z = jax.nn.sigmoid(gi[:, HID:2 * HID] + gh[:, HID:2 * HID])
    nn_ = jnp.tanh(gi[:, 2 * HID:] + r * gh[:, 2 * HID:])
    out_ref[...] = (1.0 - z) * nn_ + z * hb


def _gru(x, h, partials, counts, wixt, wict, whht, bih, bhh):
    return pl.pallas_call(
        _gru_body,
        grid=(GGRID,),
        in_specs=[
            pl.BlockSpec((GBR, HID), lambda i: (i, 0)),
            pl.BlockSpec((GBR, HID), lambda i: (i, 0)),
            pl.BlockSpec((NC, GBR, HID), lambda i: (0, i, 0)),
            pl.BlockSpec((NC, NS, GBR), lambda i: (0, 0, i)),
            pl.BlockSpec((HID, 3 * HID), lambda i: (0, 0)),
            pl.BlockSpec((HID, 3 * HID), lambda i: (0, 0)),
            pl.BlockSpec((HID, 3 * HID), lambda i: (0, 0)),
            pl.BlockSpec((1, 3 * HID), lambda i: (0, 0)),
            pl.BlockSpec((1, 3 * HID), lambda i: (0, 0)),
        ],
        out_specs=pl.BlockSpec((GBR, HID), lambda i: (i, 0)),
        out_shape=jax.ShapeDtypeStruct((N_NODES, HID), jnp.float32),
    )(x, h, partials, counts, wixt, wict, whht, bih, bhh)


def kernel(x, h, edge_index, W_msg, b_msg, W_ih, W_hh, b_ih, b_hh):
    src = edge_index[0].astype(jnp.int32)
    dst = edge_index[1].astype(jnp.int32)
    pk = lax.bitwise_or(lax.shift_left(src, 14), dst)

    wxt = W_msg[:, :HID].T
    wht = W_msg[:, HID:].T
    bm = b_msg[None, :]
    wixt = W_ih[:, :HID].T
    wict = W_ih[:, HID:].T
    whht = W_hh.T
    bih = b_ih[None, :]
    bhh = b_hh[None, :]

    za = jnp.zeros((ROWS_PER_SUB, HID), jnp.float32)
    zc = jnp.zeros((N_PAD,), jnp.float32)

    m = _prep(x, h, wxt, wht, bm)
    partials, counts = _segment_accumulate(m, pk, za, zc)
    return _gru(x, h, partials, counts, wixt, wict, whht, bih, bhh)
